# Initial kernel scaffold; baseline (speedup 1.0000x reference)
import jax
import jax.numpy as jnp
from jax.experimental import pallas as pl


def kernel(actions, action_embeddings):
    raise NotImplementedError("write your pallas kernel here")


# speedup vs baseline: 1.4274x; 1.4274x over previous
import functools

import jax
import jax.numpy as jnp
from jax import lax
from jax.experimental import pallas as pl
from jax.experimental.pallas import tpu as pltpu
from jax.experimental.pallas import tpu_sc as plsc

NC, NS = 2, 16
NW = NC * NS
BATCH, HIST, D = 4096, 50, 256
B = BATCH * HIST
CHUNK = 128
CPW = B // (NW * CHUNK)


@functools.partial(
    pl.kernel,
    out_type=jax.ShapeDtypeStruct((B, D), jnp.float32),
    mesh=plsc.VectorSubcoreMesh(core_axis_name="c", subcore_axis_name="s"),
    scratch_types=[
        pltpu.VMEM((CPW, CHUNK), jnp.int32),
        pltpu.VMEM((CHUNK, D), jnp.float32),
        pltpu.SemaphoreType.DMA,
    ],
)
def _gather_kernel(table_hbm, idx_hbm, out_hbm, idx_v, rows_v, gsem):
    wid = lax.axis_index("s") * NC + lax.axis_index("c")
    base = wid * (CPW * CHUNK)
    pltpu.sync_copy(idx_hbm.at[wid], idx_v)

    def body(c, _):
        pltpu.async_copy(table_hbm.at[idx_v.at[c]], rows_v, gsem).wait()
        pltpu.sync_copy(rows_v, out_hbm.at[pl.ds(base + c * CHUNK, CHUNK)])
        return _

    lax.fori_loop(0, CPW, body, 0)


def kernel(actions, action_embeddings):
    idx = actions.reshape(NW, CPW, CHUNK).astype(jnp.int32)
    out = _gather_kernel(action_embeddings, idx)
    return out.reshape(BATCH, HIST, D)
